# baseline (speedup 1.0000x reference)
import jax
import jax.numpy as jnp
from jax import lax
from jax.experimental import pallas as pl
from jax.experimental.pallas import tpu as pltpu

_NUM_EXPERTS = 64
_SUB = 512
_RING = 4


def _top2(logits):
    e = logits.shape[-1]
    iota = jax.lax.broadcasted_iota(jnp.int32, logits.shape, 1)
    m1 = jnp.max(logits, axis=-1, keepdims=True)
    i1 = jnp.min(jnp.where(logits == m1, iota, e), axis=-1, keepdims=True)
    masked = jnp.where(iota == i1, -jnp.inf, logits)
    m2 = jnp.max(masked, axis=-1, keepdims=True)
    i2 = jnp.min(jnp.where(masked == m2, iota, e), axis=-1, keepdims=True)
    z = jnp.sum(jnp.exp(logits - m1), axis=-1, keepdims=True)
    idx = jnp.concatenate([i1, i2], axis=-1)
    score = jnp.concatenate([1.0 / z, jnp.exp(m2 - m1) / z], axis=-1)
    return idx, score


def _router_kernel(x_hbm, w_ref, b_ref, idx_ref, score_ref, buf, sems):
    nsub = x_hbm.shape[0] // _SUB

    def copy(i, slot):
        return pltpu.make_async_copy(
            x_hbm.at[pl.ds(i * _SUB, _SUB), :], buf.at[slot], sems.at[slot]
        )

    for s in range(_RING):
        copy(s, s).start()

    def body(i, carry):
        slot = lax.bitwise_and(i, _RING - 1)
        copy(i, slot).wait()
        logits = jax.lax.dot_general(
            buf[slot], w_ref[...], (((1,), (1,)), ((), ())),
            preferred_element_type=jnp.float32,
        ) + b_ref[...]

        @pl.when(i + _RING < nsub)
        def _():
            copy(i + _RING, slot).start()

        idx, score = _top2(logits)
        idx_ref[pl.ds(i * _SUB, _SUB), :] = idx
        score_ref[pl.ds(i * _SUB, _SUB), :] = score
        return carry

    lax.fori_loop(0, nsub, body, 0)


@jax.jit
def kernel(x, W, b):
    tokens, d_model = x.shape
    b2 = b.reshape(1, _NUM_EXPERTS)
    idx, scores = pl.pallas_call(
        _router_kernel,
        in_specs=[
            pl.BlockSpec(memory_space=pltpu.HBM),
            pl.BlockSpec((_NUM_EXPERTS, d_model), lambda: (0, 0)),
            pl.BlockSpec((1, _NUM_EXPERTS), lambda: (0, 0)),
        ],
        out_specs=[
            pl.BlockSpec((tokens, 2), lambda: (0, 0)),
            pl.BlockSpec((tokens, 2), lambda: (0, 0)),
        ],
        out_shape=[
            jax.ShapeDtypeStruct((tokens, 2), jnp.int32),
            jax.ShapeDtypeStruct((tokens, 2), jnp.float32),
        ],
        scratch_shapes=[
            pltpu.VMEM((_RING, _SUB, d_model), jnp.float32),
            pltpu.SemaphoreType.DMA((_RING,)),
        ],
    )(x, W, b2)
    return idx, scores
